# baseline (speedup 1.0000x reference)
import jax
import jax.numpy as jnp
from jax import lax
from jax.experimental import pallas as pl
from jax.experimental.pallas import tpu as pltpu
from jax.experimental.pallas import tpu_sc as plsc

N = 10000
E = 320000
D = 128
NP = 10240
DR = NP // 16

NC = 2
NS = 16
NW = NC * NS
EPW = E // NW
G = 80
NG = EPW // G
NSG = 5
GSG = NG // NSG
NCH = NP // G
CPS = NCH // NS
DRS = DR // NS


def _sc_body(x_hbm, src_hbm, dst_hbm,
             agg_hbm, deg_hbm,
             src_v, dst_v, rows_v, deg_v, idx_v, acc_sh, degsh, sem, sem2):
    c = lax.axis_index("c")
    s = lax.axis_index("s")
    w = c * NS + s

    zeros16 = jnp.zeros((16,), jnp.float32)
    ones16 = jnp.ones((16,), jnp.float32)
    iota16 = lax.iota(jnp.int32, 16)

    zbuf = rows_v.at[0]

    def zero_deg(r, _):
        deg_v[r] = zeros16
        return 0
    lax.fori_loop(0, DR, zero_deg, 0)
    pltpu.sync_copy(deg_v.at[pl.ds(s * DRS, DRS), :],
                    degsh.at[pl.ds(s * DRS, DRS), :])

    def zero_rows(i, _):
        for k in range(D // 16):
            zbuf[i, pl.ds(k * 16, 16)] = zeros16
        return 0
    lax.fori_loop(0, G, zero_rows, 0)

    def zero_acc(j, _):
        ch = s * CPS + j
        pltpu.sync_copy(zbuf, acc_sh.at[pl.ds(ch * G, G), :])
        return 0
    lax.fori_loop(0, CPS, zero_acc, 0)

    for r in range(NSG):
        for k in range(8):
            idx_v[r, pl.ds(k * 16, 16)] = r * 128 + k * 16 + iota16

    plsc.subcore_barrier()

    rows_a = rows_v.at[0]
    rows_b = rows_v.at[1]

    def start(g, buf, s_):
        pltpu.async_copy(x_hbm.at[src_v.at[g]], buf, s_)
        for k in range(G // 16):
            d16 = dst_v[g, pl.ds(k * 16, 16)]
            plsc.addupdate_scatter(
                deg_v, [lax.shift_right_logical(d16, 4),
                        lax.bitwise_and(d16, 15)], ones16)

    def finish(g, buf, s_):
        pltpu.make_async_copy(x_hbm.at[src_v.at[g]], buf, s_).wait()
        pltpu.sync_copy(buf, acc_sh.at[dst_v.at[g]], add=True)

    for sg in range(NSG):
        pltpu.sync_copy(src_hbm.at[w, pl.ds(sg * GSG, GSG), :], src_v)
        pltpu.sync_copy(dst_hbm.at[w, pl.ds(sg * GSG, GSG), :], dst_v)
        start(0, rows_a, sem)

        def pair(i, _):
            g = 2 * i
            start(g + 1, rows_b, sem2)
            finish(g, rows_a, sem)
            start(g + 2, rows_a, sem)
            finish(g + 1, rows_b, sem2)
            return 0
        lax.fori_loop(0, (GSG - 1) // 2, pair, 0)
        finish(GSG - 1, rows_a, sem)

    plsc.subcore_barrier()

    for r in range(NSG):
        pltpu.sync_copy(deg_v.at[pl.ds(r * 128, 128), :],
                        degsh.at[idx_v.at[r]], add=True)

    plsc.subcore_barrier()

    def write_acc(j, _):
        ch = s * CPS + j
        pltpu.sync_copy(acc_sh.at[pl.ds(ch * G, G), :], zbuf)
        pltpu.sync_copy(zbuf, agg_hbm.at[c, pl.ds(ch * G, G), :])
        return 0
    lax.fori_loop(0, CPS, write_acc, 0)

    pltpu.sync_copy(degsh.at[pl.ds(s * DRS, DRS), :],
                    deg_v.at[pl.ds(0, DRS), :])
    pltpu.sync_copy(deg_v.at[pl.ds(0, DRS), :],
                    deg_hbm.at[c, pl.ds(s * DRS, DRS), :])


@jax.jit
def _sc_aggregate(x, src_r, dst_r):
    mesh = plsc.VectorSubcoreMesh(core_axis_name="c", subcore_axis_name="s")
    f = pl.kernel(
        _sc_body,
        out_type=(
            jax.ShapeDtypeStruct((NC, NP, D), jnp.float32),
            jax.ShapeDtypeStruct((NC, DR, 16), jnp.float32),
        ),
        mesh=mesh,
        compiler_params=pltpu.CompilerParams(
            use_tc_tiling_on_sc=False, needs_layout_passes=False),
        scratch_types=[
            pltpu.VMEM((GSG, G), jnp.int32),
            pltpu.VMEM((GSG, G), jnp.int32),
            pltpu.VMEM((2, G, D), jnp.float32),
            pltpu.VMEM((DR, 16), jnp.float32),
            pltpu.VMEM((NSG, 128), jnp.int32),
            pltpu.VMEM_SHARED((NP, D), jnp.float32),
            pltpu.VMEM_SHARED((DR, 16), jnp.float32),
            pltpu.SemaphoreType.DMA,
            pltpu.SemaphoreType.DMA,
        ],
    )
    return f(x, src_r, dst_r)


def _tc_body(agg_ref, deg_ref, x_ref, w_ref, b_ref, o_ref):
    agg = agg_ref[0] + agg_ref[1] + x_ref[...]
    deg = jnp.sum(deg_ref[...], axis=1, keepdims=True) + 1.0
    h = agg / deg
    o_ref[...] = (
        jnp.dot(h, w_ref[...], preferred_element_type=jnp.float32) + b_ref[...]
    )


def _tc_finish(agg_p, deg_p, x, W, b2):
    blk = 2048
    grid = NP // blk
    return pl.pallas_call(
        _tc_body,
        grid=(grid,),
        in_specs=[
            pl.BlockSpec((NC, blk, D), lambda i: (0, i, 0)),
            pl.BlockSpec((blk, NC), lambda i: (i, 0)),
            pl.BlockSpec((blk, D), lambda i: (i, 0)),
            pl.BlockSpec((D, D), lambda i: (0, 0)),
            pl.BlockSpec((1, D), lambda i: (0, 0)),
        ],
        out_specs=pl.BlockSpec((blk, D), lambda i: (i, 0)),
        out_shape=jax.ShapeDtypeStruct((N, D), jnp.float32),
    )(agg_p, deg_p, x, W, b2)


def kernel(x, edge_index, W, b):
    src_r = edge_index[0].reshape(NW, NG, G)
    dst_r = edge_index[1].reshape(NW, NG, G)
    agg_p, deg_p = _sc_aggregate(x, src_r, dst_r)
    deg_t = deg_p.reshape(NC, NP).T
    return _tc_finish(agg_p, deg_t, x, W, b.reshape(1, D))


# speedup vs baseline: 1.1811x; 1.1811x over previous
import jax
import jax.numpy as jnp
from jax import lax
from jax.experimental import pallas as pl
from jax.experimental.pallas import tpu as pltpu
from jax.experimental.pallas import tpu_sc as plsc

N = 10000
E = 320000
D = 128
NP = 10240
DR = NP // 16

NC = 2
NS = 16
NW = NC * NS
EPW = E // NW
G = 80
NG = EPW // G
NA = 63
NCH = N // G
CPS = -(-NCH // NS)
DRS = DR // NS


def _sc_body(x_hbm, ei_hbm,
             agg_hbm, deg_hbm,
             src_v, dst_v, rows_v, deg_v, idx_v, acc_sh, degsh,
             sem, sem2, sem3, sem4):
    c = lax.axis_index("c")
    s = lax.axis_index("s")
    w = c * NS + s

    zeros16 = jnp.zeros((16,), jnp.float32)
    ones16 = jnp.ones((16,), jnp.float32)
    iota16 = lax.iota(jnp.int32, 16)

    rows_a = rows_v.at[0]
    rows_b = rows_v.at[1]
    zbuf = rows_a

    pltpu.async_copy(ei_hbm.at[0, w, pl.ds(0, NA), :], src_v, sem3)
    pltpu.async_copy(ei_hbm.at[1, w], dst_v, sem4)

    def zero_deg(r, _):
        deg_v[r] = zeros16
        return 0
    lax.fori_loop(0, DR, zero_deg, 0)
    pltpu.sync_copy(deg_v.at[pl.ds(s * DRS, DRS), :],
                    degsh.at[pl.ds(s * DRS, DRS), :])

    def zero_rows(i, _):
        for k in range(D // 16):
            zbuf[i, pl.ds(k * 16, 16)] = zeros16
        return 0
    lax.fori_loop(0, G, zero_rows, 0)

    def zero_acc(j, _):
        ch = s + j * NS

        @pl.when(ch < NCH)
        def _():
            pltpu.sync_copy(zbuf, acc_sh.at[pl.ds(ch * G, G), :])
        return 0
    lax.fori_loop(0, CPS, zero_acc, 0)

    for r in range(5):
        for k in range(8):
            idx_v[r, pl.ds(k * 16, 16)] = r * 128 + k * 16 + iota16

    pltpu.make_async_copy(ei_hbm.at[0, w, pl.ds(0, NA), :], src_v, sem3).wait()
    pltpu.make_async_copy(ei_hbm.at[1, w], dst_v, sem4).wait()

    plsc.subcore_barrier()

    def start(g, srow, buf, s_):
        pltpu.async_copy(x_hbm.at[src_v.at[srow]], buf, s_)
        for k in range(G // 16):
            d16 = dst_v[g, pl.ds(k * 16, 16)]
            plsc.addupdate_scatter(
                deg_v, [lax.shift_right_logical(d16, 4),
                        lax.bitwise_and(d16, 15)], ones16)

    def finish(g, srow, buf, s_):
        pltpu.make_async_copy(x_hbm.at[src_v.at[srow]], buf, s_).wait()
        pltpu.sync_copy(buf, acc_sh.at[dst_v.at[g]], add=True)

    start(0, 0, rows_a, sem)

    def pair_a(i, _):
        g = 2 * i
        start(g + 1, g + 1, rows_b, sem2)
        finish(g, g, rows_a, sem)
        start(g + 2, g + 2, rows_a, sem)
        finish(g + 1, g + 1, rows_b, sem2)
        return 0
    lax.fori_loop(0, (NA - 1) // 2, pair_a, 0)
    finish(NA - 1, NA - 1, rows_a, sem)

    pltpu.sync_copy(ei_hbm.at[0, w, pl.ds(NA, NG - NA), :],
                    src_v.at[pl.ds(0, NG - NA), :])

    start(NA, 0, rows_a, sem)

    def pair_b(i, _):
        g = NA + 2 * i
        start(g + 1, g + 1 - NA, rows_b, sem2)
        finish(g, g - NA, rows_a, sem)
        start(g + 2, g + 2 - NA, rows_a, sem)
        finish(g + 1, g + 1 - NA, rows_b, sem2)
        return 0
    lax.fori_loop(0, (NG - NA) // 2 - 1, pair_b, 0)
    start(NG - 1, NG - 1 - NA, rows_b, sem2)
    finish(NG - 2, NG - 2 - NA, rows_a, sem)
    finish(NG - 1, NG - 1 - NA, rows_b, sem2)

    plsc.subcore_barrier()

    for r in range(5):
        pltpu.sync_copy(deg_v.at[pl.ds(r * 128, 128), :],
                        degsh.at[idx_v.at[r]], add=True)

    plsc.subcore_barrier()

    def wb_read(j, buf):
        ch = s + j * NS

        @pl.when(ch < NCH)
        def _():
            pltpu.sync_copy(acc_sh.at[pl.ds(ch * G, G), :], buf)

    def wb_write(j, buf, s_):
        ch = s + j * NS

        @pl.when(ch < NCH)
        def _():
            pltpu.async_copy(buf, agg_hbm.at[c, pl.ds(ch * G, G), :], s_)

    def wb_wait(j, buf, s_):
        ch = s + j * NS

        @pl.when(ch < NCH)
        def _():
            pltpu.make_async_copy(
                buf, agg_hbm.at[c, pl.ds(ch * G, G), :], s_).wait()

    wb_read(0, rows_a)
    wb_write(0, rows_a, sem)

    def wb_pair(i, _):
        j = 2 * i
        wb_read(j + 1, rows_b)
        wb_write(j + 1, rows_b, sem2)
        wb_wait(j, rows_a, sem)
        wb_read(j + 2, rows_a)
        wb_write(j + 2, rows_a, sem)
        wb_wait(j + 1, rows_b, sem2)
        return 0
    lax.fori_loop(0, CPS // 2 - 1, wb_pair, 0)
    wb_read(CPS - 1, rows_b)
    wb_write(CPS - 1, rows_b, sem2)
    wb_wait(CPS - 2, rows_a, sem)
    wb_wait(CPS - 1, rows_b, sem2)

    pltpu.sync_copy(degsh.at[pl.ds(s * DRS, DRS), :],
                    deg_v.at[pl.ds(0, DRS), :])
    pltpu.sync_copy(deg_v.at[pl.ds(0, DRS), :],
                    deg_hbm.at[c, pl.ds(s * DRS, DRS), :])


@jax.jit
def _sc_aggregate(x, ei4):
    mesh = plsc.VectorSubcoreMesh(core_axis_name="c", subcore_axis_name="s")
    f = pl.kernel(
        _sc_body,
        out_type=(
            jax.ShapeDtypeStruct((NC, N, D), jnp.float32),
            jax.ShapeDtypeStruct((NC, DR, 16), jnp.float32),
        ),
        mesh=mesh,
        compiler_params=pltpu.CompilerParams(
            use_tc_tiling_on_sc=False, needs_layout_passes=False),
        scratch_types=[
            pltpu.VMEM((NA, G), jnp.int32),
            pltpu.VMEM((NG, G), jnp.int32),
            pltpu.VMEM((2, G, D), jnp.float32),
            pltpu.VMEM((DR, 16), jnp.float32),
            pltpu.VMEM((5, 128), jnp.int32),
            pltpu.VMEM_SHARED((N, D), jnp.float32),
            pltpu.VMEM_SHARED((DR, 16), jnp.float32),
            pltpu.SemaphoreType.DMA,
            pltpu.SemaphoreType.DMA,
            pltpu.SemaphoreType.DMA,
            pltpu.SemaphoreType.DMA,
        ],
    )
    return f(x, ei4)


def _tc_body(agg_ref, deg_ref, x_ref, w_ref, b_ref, o_ref):
    agg = agg_ref[0] + agg_ref[1] + x_ref[...]
    deg = deg_ref[...]
    inv = 1.0 / (deg[0] + deg[1] + 1.0)
    h = agg * inv[:, None]
    o_ref[...] = (
        jnp.dot(h, w_ref[...], preferred_element_type=jnp.float32) + b_ref[...]
    )


def _tc_finish(agg_p, deg_p, x, W, b2):
    blk = 2048
    grid = NP // blk
    return pl.pallas_call(
        _tc_body,
        grid=(grid,),
        in_specs=[
            pl.BlockSpec((NC, blk, D), lambda i: (0, i, 0)),
            pl.BlockSpec((NC, blk), lambda i: (0, i)),
            pl.BlockSpec((blk, D), lambda i: (i, 0)),
            pl.BlockSpec((D, D), lambda i: (0, 0)),
            pl.BlockSpec((1, D), lambda i: (0, 0)),
        ],
        out_specs=pl.BlockSpec((blk, D), lambda i: (i, 0)),
        out_shape=jax.ShapeDtypeStruct((N, D), jnp.float32),
    )(agg_p, deg_p, x, W, b2)


def kernel(x, edge_index, W, b):
    ei4 = edge_index.reshape(2, NW, NG, G)
    agg_p, deg_p = _sc_aggregate(x, ei4)
    return _tc_finish(agg_p, deg_p.reshape(NC, NP), x, W, b.reshape(1, D))


# speedup vs baseline: 1.1841x; 1.0025x over previous
import jax
import jax.numpy as jnp
from jax import lax
from jax.experimental import pallas as pl
from jax.experimental.pallas import tpu as pltpu
from jax.experimental.pallas import tpu_sc as plsc

N = 10000
E = 320000
D = 128
NP = 10240
DR = NP // 16

NC = 2
NS = 16
NW = NC * NS
EPW = E // NW
G = 80
NG = EPW // G
NA = 63
NCH = N // G
CPS = -(-NCH // NS)
DRS = DR // NS


def _sc_body(x_hbm, ei_hbm,
             agg_hbm, deg_hbm,
             src_v, dst_v, rows_v, deg_v, idx_v, acc_sh, degsh,
             sem, sem2, sem3, sem4):
    c = lax.axis_index("c")
    s = lax.axis_index("s")
    w = c * NS + s

    zeros16 = jnp.zeros((16,), jnp.float32)
    ones16 = jnp.ones((16,), jnp.float32)
    iota16 = lax.iota(jnp.int32, 16)

    rows_a = rows_v.at[0]
    rows_b = rows_v.at[1]
    zbuf = rows_a

    pltpu.async_copy(ei_hbm.at[0, pl.ds(w * EPW, NA * G)], src_v, sem3)
    pltpu.async_copy(ei_hbm.at[1, pl.ds(w * EPW, EPW)], dst_v, sem4)

    def zero_deg(r, _):
        deg_v[r] = zeros16
        return 0
    lax.fori_loop(0, DR, zero_deg, 0)
    pltpu.sync_copy(deg_v.at[pl.ds(s * DRS, DRS), :],
                    degsh.at[pl.ds(s * DRS, DRS), :])

    def zero_rows(i, _):
        for k in range(D // 16):
            zbuf[i, pl.ds(k * 16, 16)] = zeros16
        return 0
    lax.fori_loop(0, G, zero_rows, 0)

    def zero_acc(j, _):
        ch = s + j * NS

        @pl.when(ch < NCH)
        def _():
            pltpu.sync_copy(zbuf, acc_sh.at[pl.ds(ch * G, G), :])
        return 0
    lax.fori_loop(0, CPS, zero_acc, 0)

    for r in range(5):
        for k in range(8):
            idx_v[r, pl.ds(k * 16, 16)] = r * 128 + k * 16 + iota16

    pltpu.make_async_copy(
        ei_hbm.at[0, pl.ds(w * EPW, NA * G)], src_v, sem3).wait()
    pltpu.make_async_copy(
        ei_hbm.at[1, pl.ds(w * EPW, EPW)], dst_v, sem4).wait()

    plsc.subcore_barrier()

    def start(g, srow, buf, s_):
        pltpu.async_copy(x_hbm.at[src_v.at[pl.ds(srow * G, G)]], buf, s_)
        for k in range(G // 16):
            d16 = dst_v[pl.ds(g * G + k * 16, 16)]
            plsc.addupdate_scatter(
                deg_v, [lax.shift_right_logical(d16, 4),
                        lax.bitwise_and(d16, 15)], ones16)

    def finish(g, srow, buf, s_):
        pltpu.make_async_copy(
            x_hbm.at[src_v.at[pl.ds(srow * G, G)]], buf, s_).wait()
        pltpu.sync_copy(buf, acc_sh.at[dst_v.at[pl.ds(g * G, G)]], add=True)

    start(0, 0, rows_a, sem)

    def pair_a(i, _):
        g = 2 * i
        start(g + 1, g + 1, rows_b, sem2)
        finish(g, g, rows_a, sem)
        start(g + 2, g + 2, rows_a, sem)
        finish(g + 1, g + 1, rows_b, sem2)
        return 0
    lax.fori_loop(0, (NA - 1) // 2, pair_a, 0)
    finish(NA - 1, NA - 1, rows_a, sem)

    pltpu.sync_copy(ei_hbm.at[0, pl.ds(w * EPW + NA * G, (NG - NA) * G)],
                    src_v.at[pl.ds(0, (NG - NA) * G)])

    start(NA, 0, rows_a, sem)

    def pair_b(i, _):
        g = NA + 2 * i
        start(g + 1, g + 1 - NA, rows_b, sem2)
        finish(g, g - NA, rows_a, sem)
        start(g + 2, g + 2 - NA, rows_a, sem)
        finish(g + 1, g + 1 - NA, rows_b, sem2)
        return 0
    lax.fori_loop(0, (NG - NA) // 2 - 1, pair_b, 0)
    start(NG - 1, NG - 1 - NA, rows_b, sem2)
    finish(NG - 2, NG - 2 - NA, rows_a, sem)
    finish(NG - 1, NG - 1 - NA, rows_b, sem2)

    plsc.subcore_barrier()

    for r in range(5):
        pltpu.sync_copy(deg_v.at[pl.ds(r * 128, 128), :],
                        degsh.at[idx_v.at[r]], add=True)

    plsc.subcore_barrier()

    def wb_read(j, buf):
        ch = s + j * NS

        @pl.when(ch < NCH)
        def _():
            pltpu.sync_copy(acc_sh.at[pl.ds(ch * G, G), :], buf)

    def wb_write(j, buf, s_):
        ch = s + j * NS

        @pl.when(ch < NCH)
        def _():
            pltpu.async_copy(buf, agg_hbm.at[c, pl.ds(ch * G, G), :], s_)

    def wb_wait(j, buf, s_):
        ch = s + j * NS

        @pl.when(ch < NCH)
        def _():
            pltpu.make_async_copy(
                buf, agg_hbm.at[c, pl.ds(ch * G, G), :], s_).wait()

    wb_read(0, rows_a)
    wb_write(0, rows_a, sem)

    def wb_pair(i, _):
        j = 2 * i
        wb_read(j + 1, rows_b)
        wb_write(j + 1, rows_b, sem2)
        wb_wait(j, rows_a, sem)
        wb_read(j + 2, rows_a)
        wb_write(j + 2, rows_a, sem)
        wb_wait(j + 1, rows_b, sem2)
        return 0
    lax.fori_loop(0, CPS // 2 - 1, wb_pair, 0)
    wb_read(CPS - 1, rows_b)
    wb_write(CPS - 1, rows_b, sem2)
    wb_wait(CPS - 2, rows_a, sem)
    wb_wait(CPS - 1, rows_b, sem2)

    pltpu.sync_copy(degsh.at[pl.ds(s * DRS, DRS), :],
                    deg_v.at[pl.ds(0, DRS), :])
    pltpu.sync_copy(deg_v.at[pl.ds(0, DRS), :],
                    deg_hbm.at[c, pl.ds(s * DRS, DRS), :])


@jax.jit
def _sc_aggregate(x, ei):
    mesh = plsc.VectorSubcoreMesh(core_axis_name="c", subcore_axis_name="s")
    f = pl.kernel(
        _sc_body,
        out_type=(
            jax.ShapeDtypeStruct((NC, N, D), jnp.float32),
            jax.ShapeDtypeStruct((NC, DR, 16), jnp.float32),
        ),
        mesh=mesh,
        compiler_params=pltpu.CompilerParams(
            use_tc_tiling_on_sc=False, needs_layout_passes=False),
        scratch_types=[
            pltpu.VMEM((NA * G,), jnp.int32),
            pltpu.VMEM((EPW,), jnp.int32),
            pltpu.VMEM((2, G, D), jnp.float32),
            pltpu.VMEM((DR, 16), jnp.float32),
            pltpu.VMEM((5, 128), jnp.int32),
            pltpu.VMEM_SHARED((N, D), jnp.float32),
            pltpu.VMEM_SHARED((DR, 16), jnp.float32),
            pltpu.SemaphoreType.DMA,
            pltpu.SemaphoreType.DMA,
            pltpu.SemaphoreType.DMA,
            pltpu.SemaphoreType.DMA,
        ],
    )
    return f(x, ei)


def _tc_body(agg_ref, deg_ref, x_ref, w_ref, b_ref, o_ref):
    agg = agg_ref[0] + agg_ref[1] + x_ref[...]
    deg = deg_ref[...]
    inv = 1.0 / (deg[0] + deg[1] + 1.0)
    h = agg * inv[:, None]
    o_ref[...] = (
        jnp.dot(h, w_ref[...], preferred_element_type=jnp.float32) + b_ref[...]
    )


def _tc_finish(agg_p, deg_p, x, W, b2):
    blk = 2048
    grid = NP // blk
    return pl.pallas_call(
        _tc_body,
        grid=(grid,),
        in_specs=[
            pl.BlockSpec((NC, blk, D), lambda i: (0, i, 0)),
            pl.BlockSpec((NC, blk), lambda i: (0, i)),
            pl.BlockSpec((blk, D), lambda i: (i, 0)),
            pl.BlockSpec((D, D), lambda i: (0, 0)),
            pl.BlockSpec((1, D), lambda i: (0, 0)),
        ],
        out_specs=pl.BlockSpec((blk, D), lambda i: (i, 0)),
        out_shape=jax.ShapeDtypeStruct((N, D), jnp.float32),
    )(agg_p, deg_p, x, W, b2)


def kernel(x, edge_index, W, b):
    agg_p, deg_p = _sc_aggregate(x, edge_index)
    return _tc_finish(agg_p, deg_p.reshape(NC, NP), x, W, b.reshape(1, D))
